# baseline (speedup 1.0000x reference)
import functools

import jax
import jax.numpy as jnp
from jax import lax
from jax.experimental import pallas as pl
from jax.experimental.pallas import tpu as pltpu
from jax.experimental.pallas import tpu_sc as plsc

N_TOK = 8192
HID = 4096
LANES = 16
NC = 2
NS = 16
NW = NC * NS
TPW = N_TOK // NW
C = 4
NCHUNK = TPW // C
DA = 3
DB = 2
PERIOD = 6
VPT = HID // LANES


def _build():
    mesh = plsc.VectorSubcoreMesh(core_axis_name="c", subcore_axis_name="s")

    @functools.partial(
        pl.kernel,
        out_type=jax.ShapeDtypeStruct((N_TOK, HID), jnp.float32),
        mesh=mesh,
        scratch_types=[
            pltpu.VMEM((NCHUNK, 16), jnp.int32),
            pltpu.VMEM((TPW + LANES,), jnp.float32),
            pltpu.VMEM((TPW + LANES,), jnp.float32),
            *[pltpu.VMEM((C, HID), jnp.float32) for _ in range(DA + DB + 2)],
            *[pltpu.SemaphoreType.DMA for _ in range(DA + DB + 2)],
        ],
    )
    def k(moe_hbm, idx_hbm, s0_hbm, s1_hbm, out_hbm, idx_v, w0_v, w1_v, *rest):
        bufs_a = rest[:DA]
        bufs_b = rest[DA:DA + DB]
        bufs_o = rest[DA + DB:DA + DB + 2]
        sems = rest[DA + DB + 2:]
        sems_a = sems[:DA]
        sems_b = sems[DA:DA + DB]
        sems_o = sems[DA + DB:DA + DB + 2]

        wid = lax.axis_index("s") * NC + lax.axis_index("c")
        base = wid * TPW
        pltpu.sync_copy(idx_hbm.at[wid], idx_v)
        pltpu.sync_copy(s0_hbm.at[wid], w0_v.at[pl.ds(0, TPW)])
        pltpu.sync_copy(s1_hbm.at[wid], w1_v.at[pl.ds(0, TPW)])

        def norm_body(i, carry):
            sl = pl.ds(i * LANES, LANES)
            a = w0_v[sl]
            b = w1_v[sl]
            t = a + b
            w0_v[sl] = a / t
            w1_v[sl] = b / t
            return carry

        lax.fori_loop(0, TPW // LANES, norm_body, 0)

        def start_a(c, pa):
            pltpu.async_copy(moe_hbm.at[idx_v.at[c, pl.ds(0, C)]],
                             bufs_a[pa], sems_a[pa])

        def start_b(c, pb):
            pltpu.async_copy(moe_hbm.at[idx_v.at[c, pl.ds(8, C)]],
                             bufs_b[pb], sems_b[pb])

        def wait_a(pa):
            pltpu.make_async_copy(moe_hbm.at[idx_v.at[0, pl.ds(0, C)]],
                                  bufs_a[pa], sems_a[pa]).wait()

        def wait_b(pb):
            pltpu.make_async_copy(moe_hbm.at[idx_v.at[0, pl.ds(8, C)]],
                                  bufs_b[pb], sems_b[pb]).wait()

        def wait_out(pb):
            pltpu.make_async_copy(bufs_o[pb], out_hbm.at[pl.ds(base, C)],
                                  sems_o[pb]).wait()

        def compute(c, pa, pb):
            w0c = w0_v[pl.ds(c * C, LANES)]
            w1c = w1_v[pl.ds(c * C, LANES)]
            w0s = [jnp.full((LANES,), w0c[t]) for t in range(C)]
            w1s = [jnp.full((LANES,), w1c[t]) for t in range(C)]
            buf_a = bufs_a[pa]
            buf_b = bufs_b[pb]
            buf_o = bufs_o[pb]

            def vec_body(v, carry2):
                sl = pl.ds(v * LANES, LANES)
                for t in range(C):
                    buf_o[t, sl] = buf_a[t, sl] * w0s[t] + buf_b[t, sl] * w1s[t]
                return carry2

            lax.fori_loop(0, VPT, vec_body, 0, unroll=2)
            pltpu.async_copy(buf_o, out_hbm.at[pl.ds(base + c * C, C)],
                             sems_o[pb])

        start_a(0, 0)
        start_a(1, 1)
        start_b(0, 0)

        n_main = (NCHUNK // PERIOD) * PERIOD - 2

        def outer(g, carry):
            for p in range(PERIOD):
                c = g * PERIOD + p
                pa = p % DA
                pb = p % DB
                start_a(c + 2, (pa + 2) % DA)
                start_b(c + 1, (pb + 1) % DB)
                wait_a(pa)
                wait_b(pb)
                if p < 2:
                    @pl.when(c >= 2)
                    def _():
                        wait_out(pb)
                else:
                    wait_out(pb)
                compute(c, pa, pb)
            return carry

        lax.fori_loop(0, n_main // PERIOD, outer, 0)
        for c in range(n_main - n_main % PERIOD, NCHUNK):
            pa = c % DA
            pb = c % DB
            if c + 2 < NCHUNK:
                start_a(c + 2, (pa + 2) % DA)
            if c + 1 < NCHUNK:
                start_b(c + 1, (pb + 1) % DB)
            wait_a(pa)
            wait_b(pb)
            if c >= 2:
                wait_out(pb)
            compute(c, pa, pb)
        wait_out(0)
        wait_out(1)

    return k


_sc_combine = _build()


def kernel(moe_output, scores, mapped_slots, expert_counts):
    del expert_counts
    ms = mapped_slots.reshape(NW, NCHUNK, C, 2)
    idx = jnp.zeros((NW, NCHUNK, 16), jnp.int32)
    idx = idx.at[:, :, 0:C].set(ms[..., 0])
    idx = idx.at[:, :, 8:8 + C].set(ms[..., 1])
    s0 = scores[:, 0].reshape(NW, TPW)
    s1 = scores[:, 1].reshape(NW, TPW)
    return _sc_combine(moe_output, idx, s0, s1)


# speedup vs baseline: 1.0359x; 1.0359x over previous
import functools

import jax
import jax.numpy as jnp
from jax import lax
from jax.experimental import pallas as pl
from jax.experimental.pallas import tpu as pltpu
from jax.experimental.pallas import tpu_sc as plsc

N_TOK = 8192
HID = 4096
LANES = 16
NC = 2
NS = 16
NW = NC * NS
TPW = N_TOK // NW
C = 4
NCHUNK = TPW // C
VPT = HID // LANES


def _build():
    mesh = plsc.VectorSubcoreMesh(core_axis_name="c", subcore_axis_name="s")

    @functools.partial(
        pl.kernel,
        out_type=jax.ShapeDtypeStruct((N_TOK, HID), jnp.float32),
        mesh=mesh,
        scratch_types=[
            pltpu.VMEM((NCHUNK, C), jnp.int32),
            pltpu.VMEM((NCHUNK, C), jnp.int32),
            pltpu.VMEM((TPW + LANES,), jnp.float32),
            pltpu.VMEM((TPW + LANES,), jnp.float32),
            pltpu.VMEM((C, HID), jnp.float32),
            pltpu.VMEM((C, HID), jnp.float32),
            pltpu.VMEM((C, HID), jnp.float32),
            pltpu.VMEM((C, HID), jnp.float32),
            pltpu.VMEM((C, HID), jnp.float32),
            pltpu.VMEM((C, HID), jnp.float32),
            pltpu.SemaphoreType.DMA,
            pltpu.SemaphoreType.DMA,
            pltpu.SemaphoreType.DMA,
            pltpu.SemaphoreType.DMA,
        ],
    )
    def k(moe_hbm, idx0_hbm, idx1_hbm, s0_hbm, s1_hbm, out_hbm,
          idx0_v, idx1_v, w0_v, w1_v, a0, a1, b0, b1, o0, o1,
          sem_i0, sem_i1, sem_o0, sem_o1):
        wid = lax.axis_index("s") * NC + lax.axis_index("c")
        base = wid * TPW
        pltpu.sync_copy(idx0_hbm.at[wid], idx0_v)
        pltpu.sync_copy(idx1_hbm.at[wid], idx1_v)
        pltpu.sync_copy(s0_hbm.at[wid], w0_v.at[pl.ds(0, TPW)])
        pltpu.sync_copy(s1_hbm.at[wid], w1_v.at[pl.ds(0, TPW)])

        bufs_a = (a0, a1)
        bufs_b = (b0, b1)
        bufs_o = (o0, o1)
        sems_i = (sem_i0, sem_i1)
        sems_o = (sem_o0, sem_o1)

        def norm_body(i, carry):
            sl = pl.ds(i * LANES, LANES)
            a = w0_v[sl]
            b = w1_v[sl]
            t = a + b
            w0_v[sl] = a / t
            w1_v[sl] = b / t
            return carry

        lax.fori_loop(0, TPW // LANES, norm_body, 0)

        def start_gather(c, p):
            pltpu.async_copy(moe_hbm.at[idx0_v.at[c]], bufs_a[p], sems_i[p])
            pltpu.async_copy(moe_hbm.at[idx1_v.at[c]], bufs_b[p], sems_i[p])

        def wait_gather(p):
            pltpu.make_async_copy(moe_hbm.at[idx0_v.at[0]], bufs_a[p], sems_i[p]).wait()
            pltpu.make_async_copy(moe_hbm.at[idx1_v.at[0]], bufs_b[p], sems_i[p]).wait()

        def wait_out(p):
            pltpu.make_async_copy(bufs_o[p], out_hbm.at[pl.ds(base, C)], sems_o[p]).wait()

        start_gather(0, 0)

        def outer(g, carry):
            for p in range(2):
                c = g * 2 + p
                if p == 0:
                    start_gather(c + 1, 1)
                else:
                    @pl.when(g + 1 < NCHUNK // 2)
                    def _():
                        start_gather(c + 1, 0)
                wait_gather(p)

                @pl.when(g >= 1)
                def _():
                    wait_out(p)

                w0c = w0_v[pl.ds(c * C, LANES)]
                w1c = w1_v[pl.ds(c * C, LANES)]
                w0s = [jnp.full((LANES,), w0c[t]) for t in range(C)]
                w1s = [jnp.full((LANES,), w1c[t]) for t in range(C)]
                buf_a = bufs_a[p]
                buf_b = bufs_b[p]
                buf_o = bufs_o[p]

                def vec_body(v, carry2):
                    sl = pl.ds(v * LANES, LANES)
                    for t in range(C):
                        buf_o[t, sl] = buf_a[t, sl] * w0s[t] + buf_b[t, sl] * w1s[t]
                    return carry2

                lax.fori_loop(0, VPT, vec_body, 0, unroll=2)
                pltpu.async_copy(buf_o, out_hbm.at[pl.ds(base + c * C, C)], sems_o[p])
            return carry

        lax.fori_loop(0, NCHUNK // 2, outer, 0)
        wait_out(0)
        wait_out(1)

    return k


_sc_combine = _build()


def kernel(moe_output, scores, mapped_slots, expert_counts):
    del expert_counts
    idx0 = mapped_slots[:, 0].reshape(NW, NCHUNK, C)
    idx1 = mapped_slots[:, 1].reshape(NW, NCHUNK, C)
    s0 = scores[:, 0].reshape(NW, TPW)
    s1 = scores[:, 1].reshape(NW, TPW)
    return _sc_combine(moe_output, idx0, idx1, s0, s1)


# speedup vs baseline: 1.0402x; 1.0042x over previous
import functools

import jax
import jax.numpy as jnp
from jax import lax
from jax.experimental import pallas as pl
from jax.experimental.pallas import tpu as pltpu
from jax.experimental.pallas import tpu_sc as plsc

N_TOK = 8192
HID = 4096
LANES = 16
NC = 2
NS = 16
NW = NC * NS
TPW = N_TOK // NW
C = 4
NCHUNK = TPW // C
VPT = HID // LANES


def _build():
    mesh = plsc.VectorSubcoreMesh(core_axis_name="c", subcore_axis_name="s")

    @functools.partial(
        pl.kernel,
        out_type=jax.ShapeDtypeStruct((N_TOK, HID), jnp.float32),
        mesh=mesh,
        scratch_types=[
            pltpu.VMEM((NCHUNK, C), jnp.int32),
            pltpu.VMEM((NCHUNK, C), jnp.int32),
            pltpu.VMEM((TPW + LANES,), jnp.float32),
            pltpu.VMEM((TPW + LANES,), jnp.float32),
            pltpu.VMEM((C, HID), jnp.float32),
            pltpu.VMEM((C, HID), jnp.float32),
            pltpu.VMEM((C, HID), jnp.float32),
            pltpu.VMEM((C, HID), jnp.float32),
            pltpu.VMEM((C, HID), jnp.float32),
            pltpu.VMEM((C, HID), jnp.float32),
            pltpu.SemaphoreType.DMA,
            pltpu.SemaphoreType.DMA,
            pltpu.SemaphoreType.DMA,
            pltpu.SemaphoreType.DMA,
        ],
    )
    def k(moe_hbm, idx0_hbm, idx1_hbm, s0_hbm, s1_hbm, out_hbm,
          idx0_v, idx1_v, w0_v, w1_v, a0, a1, b0, b1, o0, o1,
          sem_i0, sem_i1, sem_o0, sem_o1):
        wid = lax.axis_index("s") * NC + lax.axis_index("c")
        base = wid * TPW

        bufs_a = (a0, a1)
        bufs_b = (b0, b1)
        bufs_o = (o0, o1)
        sems_i = (sem_i0, sem_i1)
        sems_o = (sem_o0, sem_o1)

        def start_gather(c, p):
            pltpu.async_copy(moe_hbm.at[idx0_v.at[c]], bufs_a[p], sems_i[p])
            pltpu.async_copy(moe_hbm.at[idx1_v.at[c]], bufs_b[p], sems_i[p])

        def wait_gather(p):
            pltpu.make_async_copy(moe_hbm.at[idx0_v.at[0]], bufs_a[p], sems_i[p]).wait()
            pltpu.make_async_copy(moe_hbm.at[idx1_v.at[0]], bufs_b[p], sems_i[p]).wait()

        def wait_out(p):
            pltpu.make_async_copy(bufs_o[p], out_hbm.at[pl.ds(base, C)], sems_o[p]).wait()

        pltpu.sync_copy(idx0_hbm.at[wid], idx0_v)
        pltpu.sync_copy(idx1_hbm.at[wid], idx1_v)
        start_gather(0, 0)
        start_gather(1, 1)
        pltpu.sync_copy(s0_hbm.at[wid], w0_v.at[pl.ds(0, TPW)])
        pltpu.sync_copy(s1_hbm.at[wid], w1_v.at[pl.ds(0, TPW)])

        def norm_body(i, carry):
            sl = pl.ds(i * LANES, LANES)
            a = w0_v[sl]
            b = w1_v[sl]
            t = a + b
            w0_v[sl] = a / t
            w1_v[sl] = b / t
            return carry

        lax.fori_loop(0, TPW // LANES, norm_body, 0)

        def outer(g, carry):
            for p in range(2):
                c = g * 2 + p
                if p == 0:
                    @pl.when(g >= 1)
                    def _():
                        start_gather(c + 1, 1)
                else:
                    @pl.when(g + 1 < NCHUNK // 2)
                    def _():
                        start_gather(c + 1, 0)
                wait_gather(p)

                @pl.when(g >= 1)
                def _():
                    wait_out(p)

                w0c = w0_v[pl.ds(c * C, LANES)]
                w1c = w1_v[pl.ds(c * C, LANES)]
                w0s = [jnp.full((LANES,), w0c[t]) for t in range(C)]
                w1s = [jnp.full((LANES,), w1c[t]) for t in range(C)]
                buf_a = bufs_a[p]
                buf_b = bufs_b[p]
                buf_o = bufs_o[p]

                def vec_body(v, carry2):
                    sl = pl.ds(v * LANES, LANES)
                    for t in range(C):
                        buf_o[t, sl] = buf_a[t, sl] * w0s[t] + buf_b[t, sl] * w1s[t]
                    return carry2

                lax.fori_loop(0, VPT, vec_body, 0, unroll=2)
                pltpu.async_copy(buf_o, out_hbm.at[pl.ds(base + c * C, C)], sems_o[p])
            return carry

        lax.fori_loop(0, NCHUNK // 2, outer, 0)
        wait_out(0)
        wait_out(1)

    return k


_sc_combine = _build()


def kernel(moe_output, scores, mapped_slots, expert_counts):
    del expert_counts
    idx0 = mapped_slots[:, 0].reshape(NW, NCHUNK, C)
    idx1 = mapped_slots[:, 1].reshape(NW, NCHUNK, C)
    s0 = scores[:, 0].reshape(NW, TPW)
    s1 = scores[:, 1].reshape(NW, TPW)
    return _sc_combine(moe_output, idx0, idx1, s0, s1)
